# baseline (speedup 1.0000x reference)
import functools

import jax
import jax.numpy as jnp
from jax import lax
from jax.experimental import pallas as pl
from jax.experimental.pallas import tpu as pltpu
from jax.experimental.pallas import tpu_sc as plsc

B = 16384
D = 64
NC = 2
BPC = B // NC

_smesh = plsc.ScalarSubcoreMesh(axis_name="c")


@functools.partial(
    pl.kernel,
    mesh=_smesh,
    out_type=jax.ShapeDtypeStruct((B, D), jnp.float32),
    scratch_types=[
        pltpu.SMEM((BPC,), jnp.int32),
        pltpu.SemaphoreType.DMA,
        pltpu.SemaphoreType.DMA,
        pltpu.SemaphoreType.DMA,
    ],
    compiler_params=pltpu.CompilerParams(use_tc_tiling_on_sc=True),
)
def _sc_gather(table_hbm, idx_hbm, out_hbm, idx_s, sem_i, sem_a, sem_b):
    cid = lax.axis_index("c")
    base = cid * BPC
    pltpu.async_copy(idx_hbm.at[pl.ds(base, BPC)], idx_s, sem_i).wait()

    @pl.loop(0, BPC, step=8)
    def _issue(i):
        for k in range(8):
            sem = sem_a if k % 2 == 0 else sem_b
            r = idx_s[i + k]
            pltpu.async_copy(
                table_hbm.at[pl.ds(r, 1)], out_hbm.at[pl.ds(base + i + k, 1)], sem
            )

    pltpu.make_async_copy(
        table_hbm.at[pl.ds(0, BPC // 2)], out_hbm.at[pl.ds(base, BPC // 2)], sem_a
    ).wait()
    pltpu.make_async_copy(
        table_hbm.at[pl.ds(0, BPC // 2)], out_hbm.at[pl.ds(base, BPC // 2)], sem_b
    ).wait()


def _blend_body(looked_ref, nv_ref, m_ref, w_ref, b_ref, out_ref):
    num = nv_ref[...] * w_ref[...] + b_ref[...]
    m = m_ref[...]
    out_ref[...] = m * num + (1.0 - m) * looked_ref[...]


_GRID = 8
_BLK = B // _GRID


def _blend(looked, nv, m, w, b):
    return pl.pallas_call(
        _blend_body,
        grid=(_GRID,),
        in_specs=[
            pl.BlockSpec((_BLK, D), lambda i: (i, 0)),
            pl.BlockSpec((_BLK, 1), lambda i: (i, 0)),
            pl.BlockSpec((_BLK, 1), lambda i: (i, 0)),
            pl.BlockSpec((1, D), lambda i: (0, 0)),
            pl.BlockSpec((1, D), lambda i: (0, 0)),
        ],
        out_specs=pl.BlockSpec((_BLK, D), lambda i: (i, 0)),
        out_shape=jax.ShapeDtypeStruct((B, D), jnp.float32),
    )(looked, nv, m, w, b)


def kernel(embedding_idx, numeric_value, is_numeric, table, W, b):
    idx = embedding_idx.astype(jnp.int32)
    looked = _sc_gather(table, idx)
    nv = numeric_value.reshape(B, 1)
    m = is_numeric.astype(jnp.float32).reshape(B, 1)
    w = W.reshape(1, D)
    bb = b.reshape(1, D)
    return _blend(looked, nv, m, w, bb)


# speedup vs baseline: 1.6274x; 1.6274x over previous
import functools

import jax
import jax.numpy as jnp
from jax import lax
from jax.experimental import pallas as pl
from jax.experimental.pallas import tpu as pltpu
from jax.experimental.pallas import tpu_sc as plsc

B = 16384
D = 64
NC = 2
BPC = B // NC

NS = 16
NW = NC * NS
BPW = B // NW

_vmesh = plsc.VectorSubcoreMesh(core_axis_name="c", subcore_axis_name="s")


@functools.partial(
    pl.kernel,
    mesh=_vmesh,
    out_type=jax.ShapeDtypeStruct((B, D), jnp.float32),
    scratch_types=[
        pltpu.MemorySpace.VMEM_SHARED((B,), jnp.int32),
        pltpu.SMEM((BPW,), jnp.int32),
        pltpu.VMEM((BPW, D), jnp.float32),
        pltpu.SemaphoreType.DMA,
    ],
    compiler_params=pltpu.CompilerParams(use_tc_tiling_on_sc=True),
)
def _sc_gather(table_hbm, idx_hbm, out_hbm, idx_sh, idx_s, rows_v, sem_g):
    sid = lax.axis_index("s")
    cid = lax.axis_index("c")
    wid = sid * NC + cid
    base = wid * BPW

    @pl.when(sid == 0)
    def _load_idx():
        pltpu.sync_copy(idx_hbm, idx_sh)

    plsc.subcore_barrier()
    pltpu.sync_copy(idx_sh.at[pl.ds(base, BPW)], idx_s)

    @pl.loop(0, BPW, step=8)
    def _issue(i):
        for k in range(8):
            r = idx_s[i + k]
            pltpu.async_copy(
                table_hbm.at[pl.ds(r, 1)], rows_v.at[pl.ds(i + k, 1)], sem_g
            )

    pltpu.make_async_copy(table_hbm.at[pl.ds(0, BPW)], rows_v, sem_g).wait()
    pltpu.sync_copy(rows_v, out_hbm.at[pl.ds(base, BPW)])


def _blend_body(looked_ref, nv_ref, m_ref, w_ref, b_ref, out_ref):
    num = nv_ref[...] * w_ref[...] + b_ref[...]
    m = m_ref[...]
    out_ref[...] = m * num + (1.0 - m) * looked_ref[...]


_GRID = 8
_BLK = B // _GRID


def _blend(looked, nv, m, w, b):
    return pl.pallas_call(
        _blend_body,
        grid=(_GRID,),
        in_specs=[
            pl.BlockSpec((_BLK, D), lambda i: (i, 0)),
            pl.BlockSpec((_BLK, 1), lambda i: (i, 0)),
            pl.BlockSpec((_BLK, 1), lambda i: (i, 0)),
            pl.BlockSpec((1, D), lambda i: (0, 0)),
            pl.BlockSpec((1, D), lambda i: (0, 0)),
        ],
        out_specs=pl.BlockSpec((_BLK, D), lambda i: (i, 0)),
        out_shape=jax.ShapeDtypeStruct((B, D), jnp.float32),
    )(looked, nv, m, w, b)


def kernel(embedding_idx, numeric_value, is_numeric, table, W, b):
    idx = embedding_idx.astype(jnp.int32)
    looked = _sc_gather(table, idx)
    nv = numeric_value.reshape(B, 1)
    m = is_numeric.astype(jnp.float32).reshape(B, 1)
    w = W.reshape(1, D)
    bb = b.reshape(1, D)
    return _blend(looked, nv, m, w, bb)


# speedup vs baseline: 1.6312x; 1.0023x over previous
import functools

import jax
import jax.numpy as jnp
from jax import lax
from jax.experimental import pallas as pl
from jax.experimental.pallas import tpu as pltpu
from jax.experimental.pallas import tpu_sc as plsc

B = 16384
D = 64
NC = 2
BPC = B // NC

NS = 16
NW = NC * NS
BPW = B // NW

_vmesh = plsc.VectorSubcoreMesh(core_axis_name="c", subcore_axis_name="s")


@functools.partial(
    pl.kernel,
    mesh=_vmesh,
    out_type=jax.ShapeDtypeStruct((B, D), jnp.float32),
    scratch_types=[
        pltpu.MemorySpace.VMEM_SHARED((B,), jnp.int32),
        pltpu.SMEM((BPW,), jnp.int32),
        pltpu.VMEM((BPW, D), jnp.float32),
        pltpu.SemaphoreType.DMA,
        pltpu.SemaphoreType.DMA,
        pltpu.SemaphoreType.DMA,
        pltpu.SemaphoreType.DMA,
    ],
    compiler_params=pltpu.CompilerParams(use_tc_tiling_on_sc=True),
)
def _sc_gather(table_hbm, idx_hbm, out_hbm, idx_sh, idx_s, rows_v, s0, s1, s2, s3):
    sid = lax.axis_index("s")
    cid = lax.axis_index("c")
    wid = sid * NC + cid
    base = wid * BPW

    pltpu.sync_copy(idx_hbm.at[pl.ds(base, BPW)], idx_sh.at[pl.ds(base, BPW)])
    pltpu.sync_copy(idx_sh.at[pl.ds(base, BPW)], idx_s)

    sems = (s0, s1, s2, s3)

    @pl.loop(0, BPW, step=8)
    def _issue(i):
        for k in range(8):
            r = idx_s[i + k]
            pltpu.async_copy(
                table_hbm.at[pl.ds(r, 1)],
                rows_v.at[pl.ds(i + k, 1)],
                sems[k % 4],
            )

    for q in range(4):
        pltpu.make_async_copy(
            table_hbm.at[pl.ds(0, BPW // 4)],
            rows_v.at[pl.ds(q * (BPW // 4), BPW // 4)],
            sems[q],
        ).wait()
    pltpu.sync_copy(rows_v, out_hbm.at[pl.ds(base, BPW)])


def _blend_body(looked_ref, nv_ref, m_ref, w_ref, b_ref, out_ref):
    num = nv_ref[...] * w_ref[...] + b_ref[...]
    m = m_ref[...]
    out_ref[...] = m * num + (1.0 - m) * looked_ref[...]


_GRID = 8
_BLK = B // _GRID


def _blend(looked, nv, m, w, b):
    return pl.pallas_call(
        _blend_body,
        grid=(_GRID,),
        in_specs=[
            pl.BlockSpec((_BLK, D), lambda i: (i, 0)),
            pl.BlockSpec((_BLK, 1), lambda i: (i, 0)),
            pl.BlockSpec((_BLK, 1), lambda i: (i, 0)),
            pl.BlockSpec((1, D), lambda i: (0, 0)),
            pl.BlockSpec((1, D), lambda i: (0, 0)),
        ],
        out_specs=pl.BlockSpec((_BLK, D), lambda i: (i, 0)),
        out_shape=jax.ShapeDtypeStruct((B, D), jnp.float32),
    )(looked, nv, m, w, b)


def kernel(embedding_idx, numeric_value, is_numeric, table, W, b):
    idx = embedding_idx.astype(jnp.int32)
    looked = _sc_gather(table, idx)
    nv = numeric_value.reshape(B, 1)
    m = is_numeric.astype(jnp.float32).reshape(B, 1)
    w = W.reshape(1, D)
    bb = b.reshape(1, D)
    return _blend(looked, nv, m, w, bb)


# speedup vs baseline: 1.6330x; 1.0011x over previous
import functools

import jax
import jax.numpy as jnp
from jax import lax
from jax.experimental import pallas as pl
from jax.experimental.pallas import tpu as pltpu
from jax.experimental.pallas import tpu_sc as plsc

B = 16384
D = 64
NC = 2
NS = 16
NW = NC * NS
BPW = B // NW

_vmesh = plsc.VectorSubcoreMesh(core_axis_name="c", subcore_axis_name="s")


@functools.partial(
    pl.kernel,
    mesh=_vmesh,
    out_type=jax.ShapeDtypeStruct((B, D), jnp.float32),
    scratch_types=[
        pltpu.MemorySpace.VMEM_SHARED((B,), jnp.int32),
        pltpu.SMEM((BPW,), jnp.int32),
        pltpu.VMEM((BPW, D), jnp.float32),
        pltpu.SemaphoreType.DMA,
    ],
    compiler_params=pltpu.CompilerParams(use_tc_tiling_on_sc=True),
)
def _sc_gather(table_hbm, idx_hbm, out_hbm, idx_sh, idx_s, rows_v, sem_g):
    sid = lax.axis_index("s")
    cid = lax.axis_index("c")
    wid = sid * NC + cid
    base = wid * BPW

    pltpu.sync_copy(idx_hbm.at[pl.ds(base, BPW)], idx_sh.at[pl.ds(base, BPW)])
    pltpu.sync_copy(idx_sh.at[pl.ds(base, BPW)], idx_s)

    @pl.loop(0, BPW, step=8)
    def _issue(i):
        for k in range(8):
            r = idx_s[i + k]

            @pl.when(r >= 0)
            def _fetch():
                pltpu.async_copy(
                    table_hbm.at[pl.ds(r, 1)], rows_v.at[pl.ds(i + k, 1)], sem_g
                )

    @pl.loop(0, BPW, step=8)
    def _drain(i):
        for k in range(8):
            r = idx_s[i + k]

            @pl.when(r >= 0)
            def _wait():
                pltpu.make_async_copy(
                    table_hbm.at[pl.ds(0, 1)], rows_v.at[pl.ds(i + k, 1)], sem_g
                ).wait()

    pltpu.sync_copy(rows_v, out_hbm.at[pl.ds(base, BPW)])


def _blend_body(looked_ref, nv_ref, m_ref, w_ref, b_ref, out_ref):
    num = nv_ref[...] * w_ref[...] + b_ref[...]
    out_ref[...] = jnp.where(m_ref[...] > 0.5, num, looked_ref[...])


_GRID = 8
_BLK = B // _GRID


def _blend(looked, nv, m, w, b):
    return pl.pallas_call(
        _blend_body,
        grid=(_GRID,),
        in_specs=[
            pl.BlockSpec((_BLK, D), lambda i: (i, 0)),
            pl.BlockSpec((_BLK, 1), lambda i: (i, 0)),
            pl.BlockSpec((_BLK, 1), lambda i: (i, 0)),
            pl.BlockSpec((1, D), lambda i: (0, 0)),
            pl.BlockSpec((1, D), lambda i: (0, 0)),
        ],
        out_specs=pl.BlockSpec((_BLK, D), lambda i: (i, 0)),
        out_shape=jax.ShapeDtypeStruct((B, D), jnp.float32),
    )(looked, nv, m, w, b)


def kernel(embedding_idx, numeric_value, is_numeric, table, W, b):
    idx = embedding_idx.astype(jnp.int32)
    midx = jnp.where(is_numeric, jnp.int32(-1), idx)
    looked = _sc_gather(table, midx)
    nv = numeric_value.reshape(B, 1)
    m = is_numeric.astype(jnp.float32).reshape(B, 1)
    w = W.reshape(1, D)
    bb = b.reshape(1, D)
    return _blend(looked, nv, m, w, bb)
